# baseline (speedup 1.0000x reference)
import functools

import jax
import jax.numpy as jnp
import numpy as np
from jax import lax
from jax.experimental import pallas as pl
from jax.experimental.pallas import tpu as pltpu
from jax.experimental.pallas import tpu_sc as plsc

N = 10000
E = 320000
D = 128
H = 8
DH = D // H
SCALE = 1.0 / np.sqrt(DH)

NC = 2
NS = 16
NW = NC * NS
C = 16
NCHUNKS = E // C
CH_PER_W = NCHUNKS // NW
SUP = 5
NSUP = CH_PER_W // SUP
NPAD = 10240
ROWS_PER_TILE = NPAD // NS
ZROWS = NPAD // 8
ZBASE = NPAD
ZROWS_PER_TILE = ZROWS // NS


_ROWB = 1000


def _qkv_body(x_ref, wq_ref, wk_ref, wv_ref, q_ref, k_ref, v_ref):
    x = x_ref[...]
    q_ref[...] = jnp.dot(x, wq_ref[...], preferred_element_type=jnp.float32,
                         precision=lax.Precision.HIGHEST)
    k_ref[...] = jnp.dot(x, wk_ref[...], preferred_element_type=jnp.float32,
                         precision=lax.Precision.HIGHEST)
    v_ref[...] = jnp.dot(x, wv_ref[...], preferred_element_type=jnp.float32,
                         precision=lax.Precision.HIGHEST)


def _qkv(x, WQ, WK, WV):
    grid = (N // _ROWB,)
    bspec_x = pl.BlockSpec((_ROWB, D), lambda i: (i, 0))
    bspec_w = pl.BlockSpec((D, D), lambda i: (0, 0))
    bspec_o = pl.BlockSpec((_ROWB, D), lambda i: (i, 0))
    return pl.pallas_call(
        _qkv_body,
        grid=grid,
        in_specs=[bspec_x, bspec_w, bspec_w, bspec_w],
        out_specs=[bspec_o, bspec_o, bspec_o],
        out_shape=[jax.ShapeDtypeStruct((N, D), jnp.float32)] * 3,
    )(x, WQ, WK, WV)


def _edge_body(q_hbm, k_hbm, v_hbm, idx_hbm,
               wv_out, z_out,
               acc_s,
               idx2, kbuf2, qbuf2, msg4, zb3, gkq, gv, gs, gi):
    cid = lax.axis_index("c")
    sid = lax.axis_index("s")
    wid = sid * NC + cid
    ch0 = wid * CH_PER_W

    def _zb(i, carry):
        for jj in range(D // DH):
            msg4[0, i, pl.ds(jj * DH, DH)] = jnp.zeros((DH,), jnp.float32)
        return carry

    lax.fori_loop(0, C, _zb, 0)

    for r in range(ROWS_PER_TILE // C):
        pltpu.sync_copy(msg4.at[0],
                        acc_s.at[pl.ds(sid * ROWS_PER_TILE + r * C, C)])
    for r in range(ZROWS_PER_TILE // C):
        pltpu.sync_copy(
            msg4.at[0],
            acc_s.at[pl.ds(ZBASE + sid * ZROWS_PER_TILE + r * C, C)])
    plsc.subcore_barrier()

    def _inv(h):
        return ((h & 1) << 3) | (((h >> 1) & 1) << 2) | (((h >> 2) & 1) << 1)

    lane = lax.iota(jnp.int32, DH)
    X8 = lane ^ 8
    X4 = lane ^ 4
    X2 = lane ^ 2
    X1 = lane ^ 1
    _hl = lane & 7
    ZPERM = (((_hl & 1) << 3) | (((_hl >> 1) & 1) << 2)
             | (((_hl >> 2) & 1) << 1))
    SPLATS = [jnp.full((DH,), _inv(h), jnp.int32) for h in range(H)]
    M8 = lane < 8
    M4 = (lane & 4) == 0
    M2 = (lane & 2) == 0
    ZERO16 = jnp.zeros((DH,), jnp.float32)

    _dnums = lax.GatherDimensionNumbers(
        offset_dims=(), collapsed_slice_dims=(0,), start_index_map=(0,))

    def _take(x, idx):
        return lax.gather(x, idx[:, None], _dnums, (1,),
                          mode=lax.GatherScatterMode.PROMISE_IN_BOUNDS)

    SUPW = 256

    def _issue_super(b):
        pltpu.async_copy(idx_hbm.at[pl.ds((ch0 + b * SUP) * 3 * C, SUPW)],
                         idx2.at[pl.ds(lax.rem(b, 3) * SUPW, SUPW)], gi)

    def _wait_super():
        pltpu.make_async_copy(idx_hbm.at[pl.ds(0, SUPW)],
                              idx2.at[pl.ds(0, SUPW)], gi).wait()

    def _issue_gathers(j):
        b = j // SUP
        bi = j - b * SUP
        sl = lax.rem(b, 3)
        s = j & 1
        src_ix = idx2.at[pl.ds(sl * SUPW + bi * 48, C)]
        dst_ix = idx2.at[pl.ds(sl * SUPW + bi * 48 + C, C)]
        pltpu.async_copy(k_hbm.at[src_ix], kbuf2.at[s], gkq.at[s])
        pltpu.async_copy(q_hbm.at[dst_ix], qbuf2.at[s], gkq.at[s])
        pltpu.async_copy(v_hbm.at[src_ix], msg4.at[j & 3], gv.at[s])

    _issue_super(0)
    _wait_super()
    _issue_super(1)
    _issue_gathers(0)
    _issue_gathers(1)

    def _chunk(j, carry):
        s = j & 1
        sm = j & 3
        zm = lax.rem(j, 3)
        b = j // SUP
        bi = j - b * SUP
        sl = lax.rem(b, 3)
        pltpu.make_async_copy(k_hbm.at[pl.ds(0, C)], kbuf2.at[s],
                              gkq.at[s]).wait()
        pltpu.make_async_copy(k_hbm.at[pl.ds(0, C)], qbuf2.at[s],
                              gkq.at[s]).wait()
        pltpu.make_async_copy(k_hbm.at[pl.ds(0, C)], msg4.at[sm],
                              gv.at[s]).wait()

        dv16 = idx2[pl.ds(sl * SUPW + bi * 48 + C, C)]
        zi16 = idx2[pl.ds(sl * SUPW + bi * 48 + 2 * C, C)]

        @plsc.parallel_loop(0, C, 1, unroll=4)
        def _edge(i):
            p = [kbuf2[s, i, pl.ds(h * DH, DH)] * qbuf2[s, i, pl.ds(h * DH, DH)]
                 for h in range(H)]
            a = [ph + _take(ph, X8) for ph in p]
            b_ = [jnp.where(M8, a[2 * k], a[2 * k + 1]) for k in range(4)]
            c = [bk + _take(bk, X4) for bk in b_]
            d = [jnp.where(M4, c[0], c[1]), jnp.where(M4, c[2], c[3])]
            e = [dm + _take(dm, X2) for dm in d]
            f = jnp.where(M2, e[0], e[1])
            g = f + _take(f, X1)
            ev = jnp.exp(jnp.clip(g * SCALE, -5.0, 5.0))
            zvec = _take(ev, ZPERM)
            dvi = _take(dv16, jnp.full((DH,), i, jnp.int32))
            dcol = dvi & 7
            zvi = lax.bitcast_convert_type(zvec, jnp.int32)
            for jj in range(8):
                m = ((dcol ^ jj) - 1) >> 31
                zb3[zm, i, pl.ds(jj * DH, DH)] = lax.bitcast_convert_type(
                    zvi & m, jnp.float32)
            for h in range(H):
                slh = pl.ds(h * DH, DH)
                msg4[sm, i, slh] = msg4[sm, i, slh] * _take(ev, SPLATS[h])

        @pl.when(j >= 2)
        def _wait_scat():
            pltpu.make_async_copy(msg4.at[0], acc_s.at[pl.ds(0, C)],
                                  gs.at[s]).wait()
            pltpu.make_async_copy(zb3.at[0], acc_s.at[pl.ds(0, C)],
                                  gs.at[s]).wait()

        pltpu.async_copy(msg4.at[sm], acc_s.at[dv16], gs.at[s], add=True)
        pltpu.async_copy(zb3.at[zm], acc_s.at[zi16], gs.at[s], add=True)

        @pl.when(j < CH_PER_W - 2)
        def _prefetch():
            jn = j + 2
            bn = jn // SUP

            @pl.when(jn - bn * SUP == 0)
            def _ld():
                _wait_super()

                @pl.when(bn + 1 < NSUP)
                def _nx():
                    _issue_super(bn + 1)

            _issue_gathers(jn)

        return carry

    lax.fori_loop(0, CH_PER_W, _chunk, 0)

    for jj in (CH_PER_W - 2, CH_PER_W - 1):
        pltpu.make_async_copy(msg4.at[0], acc_s.at[pl.ds(0, C)],
                              gs.at[jj & 1]).wait()
        pltpu.make_async_copy(zb3.at[0], acc_s.at[pl.ds(0, C)],
                              gs.at[jj & 1]).wait()

    plsc.subcore_barrier()

    def _bounce(out_ref, acc_r0, out_r0, nrows):
        pltpu.sync_copy(acc_s.at[pl.ds(acc_r0, nrows)],
                        msg4.at[0, pl.ds(0, nrows)])
        pltpu.sync_copy(msg4.at[0, pl.ds(0, nrows)],
                        out_ref.at[cid, pl.ds(out_r0, nrows)])

    row0 = sid * ROWS_PER_TILE

    @pl.when(sid < NS - 1)
    def _copy_full():
        for r in range(ROWS_PER_TILE // C):
            _bounce(wv_out, row0 + r * C, row0 + r * C, C)

    @pl.when(sid == NS - 1)
    def _copy_tail():
        tail0 = (NS - 1) * ROWS_PER_TILE
        for r in range((N - tail0) // C):
            _bounce(wv_out, tail0 + r * C, tail0 + r * C, C)

    zrow0 = sid * ZROWS_PER_TILE
    for r in range(ZROWS_PER_TILE // C):
        _bounce(z_out, ZBASE + zrow0 + r * C, zrow0 + r * C, C)


@functools.partial(jax.jit)
def _edge_call(q, k, v, idx_all):
    mesh = plsc.VectorSubcoreMesh(core_axis_name="c", subcore_axis_name="s")
    fn = pl.kernel(
        _edge_body,
        mesh=mesh,
        out_type=(jax.ShapeDtypeStruct((NC, N, D), jnp.float32),
                  jax.ShapeDtypeStruct((NC, ZROWS, D), jnp.float32)),
        scratch_types=[
            pltpu.VMEM_SHARED((NPAD + ZROWS, D), jnp.float32),
            pltpu.VMEM((3 * 256,), jnp.int32),
            pltpu.VMEM((2, C, D), jnp.float32),
            pltpu.VMEM((2, C, D), jnp.float32),
            pltpu.VMEM((4, C, D), jnp.float32),
            pltpu.VMEM((3, C, D), jnp.float32),
            pltpu.SemaphoreType.DMA((2,)),
            pltpu.SemaphoreType.DMA((2,)),
            pltpu.SemaphoreType.DMA((2,)),
            pltpu.SemaphoreType.DMA,
        ],
    )
    return fn(q, k, v, idx_all)


_S_np = np.zeros((DH, D), np.float32)
for _h in range(H):
    _S_np[_h, _h * DH:(_h + 1) * DH] = 1.0


def _comb_body(wv_ref, z_ref, s_ref, o_ref):
    wv = wv_ref[0] + wv_ref[1]
    z = z_ref[0] + z_ref[1]
    zb = jnp.dot(z, s_ref[...], preferred_element_type=jnp.float32,
                 precision=lax.Precision.HIGHEST)
    o_ref[...] = wv / (zb + 1e-6)


def _combine(wvp, zp):
    grid = (N // _ROWB,)
    return pl.pallas_call(
        _comb_body,
        grid=grid,
        in_specs=[pl.BlockSpec((NC, _ROWB, D), lambda i: (0, i, 0)),
                  pl.BlockSpec((NC, _ROWB, DH), lambda i: (0, i, 0)),
                  pl.BlockSpec((DH, D), lambda i: (0, 0))],
        out_specs=pl.BlockSpec((_ROWB, D), lambda i: (i, 0)),
        out_shape=jax.ShapeDtypeStruct((N, D), jnp.float32),
    )(wvp, zp, jnp.asarray(_S_np))


def kernel(x, edge_index, WQ, WK, WV):
    q, k, v = _qkv(x, WQ, WK, WV)
    src = edge_index[0]
    dst = edge_index[1]
    zidx = (dst >> 3) + ZBASE
    idx_all = jnp.stack([src.reshape(NCHUNKS, C),
                         dst.reshape(NCHUNKS, C),
                         zidx.reshape(NCHUNKS, C)], axis=1).reshape(-1)
    idx_all = jnp.concatenate([idx_all, jnp.zeros((64,), jnp.int32)])
    wvp, zpack = _edge_call(q, k, v, idx_all)
    zp = zpack.reshape(NC, NPAD, DH)[:, :N, :]
    return _combine(wvp, zp)


# speedup vs baseline: 1.0555x; 1.0555x over previous
import functools

import jax
import jax.numpy as jnp
import numpy as np
from jax import lax
from jax.experimental import pallas as pl
from jax.experimental.pallas import tpu as pltpu
from jax.experimental.pallas import tpu_sc as plsc

N = 10000
E = 320000
D = 128
H = 8
DH = D // H
SCALE = 1.0 / np.sqrt(DH)

NC = 2
NS = 16
NW = NC * NS
C = 16
NCHUNKS = E // C
CH_PER_W = NCHUNKS // NW
SUP = 5
NSUP = CH_PER_W // SUP
NPAD = 10240
ROWS_PER_TILE = NPAD // NS
ZROWS = NPAD // 8
ZBASE = NPAD
ZROWS_PER_TILE = ZROWS // NS


_ROWB = 1000


def _qkv_body(x_ref, wq_ref, wk_ref, wv_ref, q_ref, k_ref, v_ref):
    x = x_ref[...]
    q_ref[...] = jnp.dot(x, wq_ref[...], preferred_element_type=jnp.float32,
                         precision=lax.Precision.HIGHEST)
    k_ref[...] = jnp.dot(x, wk_ref[...], preferred_element_type=jnp.float32,
                         precision=lax.Precision.HIGHEST)
    v_ref[...] = jnp.dot(x, wv_ref[...], preferred_element_type=jnp.float32,
                         precision=lax.Precision.HIGHEST)


def _qkv(x, WQ, WK, WV):
    grid = (N // _ROWB,)
    bspec_x = pl.BlockSpec((_ROWB, D), lambda i: (i, 0))
    bspec_w = pl.BlockSpec((D, D), lambda i: (0, 0))
    bspec_o = pl.BlockSpec((_ROWB, D), lambda i: (i, 0))
    return pl.pallas_call(
        _qkv_body,
        grid=grid,
        in_specs=[bspec_x, bspec_w, bspec_w, bspec_w],
        out_specs=[bspec_o, bspec_o, bspec_o],
        out_shape=[jax.ShapeDtypeStruct((N, D), jnp.float32)] * 3,
    )(x, WQ, WK, WV)


def _edge_body(q_hbm, k_hbm, v_hbm, idx_hbm,
               wv_out, z_out,
               acc_s,
               idx2, kbuf2, qbuf2, msg4, zb3, gkq, gv, gs, gi):
    cid = lax.axis_index("c")
    sid = lax.axis_index("s")
    wid = sid * NC + cid
    ch0 = wid * CH_PER_W

    def _zb(i, carry):
        for jj in range(D // DH):
            msg4[0, i, pl.ds(jj * DH, DH)] = jnp.zeros((DH,), jnp.float32)
        return carry

    lax.fori_loop(0, C, _zb, 0)

    for r in range(ROWS_PER_TILE // C):
        pltpu.sync_copy(msg4.at[0],
                        acc_s.at[pl.ds(sid * ROWS_PER_TILE + r * C, C)])
    for r in range(ZROWS_PER_TILE // C):
        pltpu.sync_copy(
            msg4.at[0],
            acc_s.at[pl.ds(ZBASE + sid * ZROWS_PER_TILE + r * C, C)])
    plsc.subcore_barrier()

    def _inv(h):
        return ((h & 1) << 3) | (((h >> 1) & 1) << 2) | (((h >> 2) & 1) << 1)

    lane = lax.iota(jnp.int32, DH)
    X8 = lane ^ 8
    X4 = lane ^ 4
    X2 = lane ^ 2
    X1 = lane ^ 1
    _hl = lane & 7
    ZPERM = (((_hl & 1) << 3) | (((_hl >> 1) & 1) << 2)
             | (((_hl >> 2) & 1) << 1))
    SPLATS = [jnp.full((DH,), _inv(h), jnp.int32) for h in range(H)]
    M8 = lane < 8
    M4 = (lane & 4) == 0
    M2 = (lane & 2) == 0
    ZERO16 = jnp.zeros((DH,), jnp.float32)

    _dnums = lax.GatherDimensionNumbers(
        offset_dims=(), collapsed_slice_dims=(0,), start_index_map=(0,))

    def _take(x, idx):
        return lax.gather(x, idx[:, None], _dnums, (1,),
                          mode=lax.GatherScatterMode.PROMISE_IN_BOUNDS)

    def _issue_super(b):
        pltpu.async_copy(idx_hbm.at[pl.ds(ch0 + b * SUP, SUP)],
                         idx2.at[lax.rem(b, 3)], gi)

    def _wait_super():
        pltpu.make_async_copy(idx_hbm.at[pl.ds(0, SUP)], idx2.at[0],
                              gi).wait()

    def _issue_gathers(j):
        b = j // SUP
        bi = j - b * SUP
        sl = lax.rem(b, 3)
        s = j & 1
        pltpu.async_copy(k_hbm.at[idx2.at[sl, bi, 0]], kbuf2.at[s], gkq.at[s])
        pltpu.async_copy(q_hbm.at[idx2.at[sl, bi, 1]], qbuf2.at[s], gkq.at[s])
        pltpu.async_copy(v_hbm.at[idx2.at[sl, bi, 0]], msg4.at[j & 3],
                         gv.at[s])

    _issue_super(0)
    _wait_super()
    _issue_super(1)
    _issue_gathers(0)
    _issue_gathers(1)

    def _chunk(j, carry):
        s = j & 1
        sm = j & 3
        zm = lax.rem(j, 3)
        b = j // SUP
        bi = j - b * SUP
        sl = lax.rem(b, 3)
        pltpu.make_async_copy(k_hbm.at[pl.ds(0, C)], kbuf2.at[s],
                              gkq.at[s]).wait()
        pltpu.make_async_copy(k_hbm.at[pl.ds(0, C)], qbuf2.at[s],
                              gkq.at[s]).wait()
        pltpu.make_async_copy(k_hbm.at[pl.ds(0, C)], msg4.at[sm],
                              gv.at[s]).wait()

        dv16 = idx2[sl, bi, 1, :]

        @plsc.parallel_loop(0, C, 1, unroll=8)
        def _edge(i):
            p = [kbuf2[s, i, pl.ds(h * DH, DH)] * qbuf2[s, i, pl.ds(h * DH, DH)]
                 for h in range(H)]
            a = [ph + _take(ph, X8) for ph in p]
            b_ = [jnp.where(M8, a[2 * k], a[2 * k + 1]) for k in range(4)]
            c = [bk + _take(bk, X4) for bk in b_]
            d = [jnp.where(M4, c[0], c[1]), jnp.where(M4, c[2], c[3])]
            e = [dm + _take(dm, X2) for dm in d]
            f = jnp.where(M2, e[0], e[1])
            g = f + _take(f, X1)
            ev = jnp.exp(jnp.clip(g * SCALE, -5.0, 5.0))
            zvec = _take(ev, ZPERM)
            dvi = _take(dv16, jnp.full((DH,), i, jnp.int32))
            dcol = dvi & 7
            zvi = lax.bitcast_convert_type(zvec, jnp.int32)
            for jj in range(8):
                m = ((dcol ^ jj) - 1) >> 31
                zb3[zm, i, pl.ds(jj * DH, DH)] = lax.bitcast_convert_type(
                    zvi & m, jnp.float32)
            for h in range(H):
                slh = pl.ds(h * DH, DH)
                msg4[sm, i, slh] = msg4[sm, i, slh] * _take(ev, SPLATS[h])

        @pl.when(j >= 2)
        def _wait_scat():
            pltpu.make_async_copy(msg4.at[0], acc_s.at[pl.ds(0, C)],
                                  gs.at[s]).wait()
            pltpu.make_async_copy(zb3.at[0], acc_s.at[pl.ds(0, C)],
                                  gs.at[s]).wait()

        pltpu.async_copy(msg4.at[sm], acc_s.at[idx2.at[sl, bi, 1]],
                         gs.at[s], add=True)
        pltpu.async_copy(zb3.at[zm], acc_s.at[idx2.at[sl, bi, 2]],
                         gs.at[s], add=True)

        @pl.when(j < CH_PER_W - 2)
        def _prefetch():
            jn = j + 2
            bn = jn // SUP

            @pl.when(jn - bn * SUP == 0)
            def _ld():
                _wait_super()

                @pl.when(bn + 1 < NSUP)
                def _nx():
                    _issue_super(bn + 1)

            _issue_gathers(jn)

        return carry

    lax.fori_loop(0, CH_PER_W, _chunk, 0)

    for jj in (CH_PER_W - 2, CH_PER_W - 1):
        pltpu.make_async_copy(msg4.at[0], acc_s.at[pl.ds(0, C)],
                              gs.at[jj & 1]).wait()
        pltpu.make_async_copy(zb3.at[0], acc_s.at[pl.ds(0, C)],
                              gs.at[jj & 1]).wait()

    plsc.subcore_barrier()

    def _bounce(out_ref, acc_r0, out_r0, nrows):
        pltpu.sync_copy(acc_s.at[pl.ds(acc_r0, nrows)],
                        msg4.at[0, pl.ds(0, nrows)])
        pltpu.sync_copy(msg4.at[0, pl.ds(0, nrows)],
                        out_ref.at[cid, pl.ds(out_r0, nrows)])

    row0 = sid * ROWS_PER_TILE

    @pl.when(sid < NS - 1)
    def _copy_full():
        for r in range(ROWS_PER_TILE // C):
            _bounce(wv_out, row0 + r * C, row0 + r * C, C)

    @pl.when(sid == NS - 1)
    def _copy_tail():
        tail0 = (NS - 1) * ROWS_PER_TILE
        for r in range((N - tail0) // C):
            _bounce(wv_out, tail0 + r * C, tail0 + r * C, C)

    zrow0 = sid * ZROWS_PER_TILE
    for r in range(ZROWS_PER_TILE // C):
        _bounce(z_out, ZBASE + zrow0 + r * C, zrow0 + r * C, C)


@functools.partial(jax.jit)
def _edge_call(q, k, v, idx_all):
    mesh = plsc.VectorSubcoreMesh(core_axis_name="c", subcore_axis_name="s")
    fn = pl.kernel(
        _edge_body,
        mesh=mesh,
        out_type=(jax.ShapeDtypeStruct((NC, N, D), jnp.float32),
                  jax.ShapeDtypeStruct((NC, ZROWS, D), jnp.float32)),
        scratch_types=[
            pltpu.VMEM_SHARED((NPAD + ZROWS, D), jnp.float32),
            pltpu.VMEM((3, SUP, 3, C), jnp.int32),
            pltpu.VMEM((2, C, D), jnp.float32),
            pltpu.VMEM((2, C, D), jnp.float32),
            pltpu.VMEM((4, C, D), jnp.float32),
            pltpu.VMEM((3, C, D), jnp.float32),
            pltpu.SemaphoreType.DMA((2,)),
            pltpu.SemaphoreType.DMA((2,)),
            pltpu.SemaphoreType.DMA((2,)),
            pltpu.SemaphoreType.DMA,
        ],
    )
    return fn(q, k, v, idx_all)


_S_np = np.zeros((DH, D), np.float32)
for _h in range(H):
    _S_np[_h, _h * DH:(_h + 1) * DH] = 1.0


def _comb_body(wv_ref, z_ref, s_ref, o_ref):
    wv = wv_ref[0] + wv_ref[1]
    z = z_ref[0] + z_ref[1]
    zb = jnp.dot(z, s_ref[...], preferred_element_type=jnp.float32,
                 precision=lax.Precision.HIGHEST)
    o_ref[...] = wv / (zb + 1e-6)


def _combine(wvp, zp):
    grid = (N // _ROWB,)
    return pl.pallas_call(
        _comb_body,
        grid=grid,
        in_specs=[pl.BlockSpec((NC, _ROWB, D), lambda i: (0, i, 0)),
                  pl.BlockSpec((NC, _ROWB, DH), lambda i: (0, i, 0)),
                  pl.BlockSpec((DH, D), lambda i: (0, 0))],
        out_specs=pl.BlockSpec((_ROWB, D), lambda i: (i, 0)),
        out_shape=jax.ShapeDtypeStruct((N, D), jnp.float32),
    )(wvp, zp, jnp.asarray(_S_np))


def kernel(x, edge_index, WQ, WK, WV):
    q, k, v = _qkv(x, WQ, WK, WV)
    src = edge_index[0]
    dst = edge_index[1]
    zidx = (dst >> 3) + ZBASE
    idx_all = (jnp.stack([src, dst, zidx], axis=0)
               .reshape(3, NCHUNKS, C).transpose(1, 0, 2))
    wvp, zpack = _edge_call(q, k, v, idx_all)
    zp = zpack.reshape(NC, NPAD, DH)[:, :N, :]
    return _combine(wvp, zp)


# speedup vs baseline: 1.0634x; 1.0076x over previous
import functools

import jax
import jax.numpy as jnp
import numpy as np
from jax import lax
from jax.experimental import pallas as pl
from jax.experimental.pallas import tpu as pltpu
from jax.experimental.pallas import tpu_sc as plsc

N = 10000
E = 320000
D = 128
H = 8
DH = D // H
SCALE = 1.0 / np.sqrt(DH)

NC = 2
NS = 16
NW = NC * NS
C = 16
NCHUNKS = E // C
CH_PER_W = NCHUNKS // NW
SUP = 5
NSUP = CH_PER_W // SUP
NPAD = 10240
ROWS_PER_TILE = NPAD // NS
ZROWS = NPAD // 8
ZBASE = NPAD
ZROWS_PER_TILE = ZROWS // NS


_ROWB = 1000


def _qkv_body(x_ref, wq_ref, wk_ref, wv_ref, q_ref, k_ref, v_ref):
    x = x_ref[...]
    q_ref[...] = jnp.dot(x, wq_ref[...], preferred_element_type=jnp.float32,
                         precision=lax.Precision.HIGHEST)
    k_ref[...] = jnp.dot(x, wk_ref[...], preferred_element_type=jnp.float32,
                         precision=lax.Precision.HIGHEST)
    v_ref[...] = jnp.dot(x, wv_ref[...], preferred_element_type=jnp.float32,
                         precision=lax.Precision.HIGHEST)


def _qkv(x, WQ, WK, WV):
    grid = (N // _ROWB,)
    bspec_x = pl.BlockSpec((_ROWB, D), lambda i: (i, 0))
    bspec_w = pl.BlockSpec((D, D), lambda i: (0, 0))
    bspec_o = pl.BlockSpec((_ROWB, D), lambda i: (i, 0))
    return pl.pallas_call(
        _qkv_body,
        grid=grid,
        in_specs=[bspec_x, bspec_w, bspec_w, bspec_w],
        out_specs=[bspec_o, bspec_o, bspec_o],
        out_shape=[jax.ShapeDtypeStruct((N, D), jnp.float32)] * 3,
    )(x, WQ, WK, WV)


def _edge_body(q_hbm, k_hbm, v_hbm, idx_hbm,
               wv_out, z_out,
               acc_s,
               idx2, kbuf2, qbuf2, msg4, zb3, gkq, gv, gs, gi):
    cid = lax.axis_index("c")
    sid = lax.axis_index("s")
    wid = sid * NC + cid
    ch0 = wid * CH_PER_W

    def _zb(i, carry):
        for jj in range(D // DH):
            msg4[0, i, pl.ds(jj * DH, DH)] = jnp.zeros((DH,), jnp.float32)
        return carry

    lax.fori_loop(0, C, _zb, 0)

    for r in range(ROWS_PER_TILE // C):
        pltpu.sync_copy(msg4.at[0],
                        acc_s.at[pl.ds(sid * ROWS_PER_TILE + r * C, C)])
    for r in range(ZROWS_PER_TILE // C):
        pltpu.sync_copy(
            msg4.at[0],
            acc_s.at[pl.ds(ZBASE + sid * ZROWS_PER_TILE + r * C, C)])
    plsc.subcore_barrier()

    def _inv(h):
        return ((h & 1) << 3) | (((h >> 1) & 1) << 2) | (((h >> 2) & 1) << 1)

    lane = lax.iota(jnp.int32, DH)
    X8 = lane ^ 8
    X4 = lane ^ 4
    X2 = lane ^ 2
    X1 = lane ^ 1
    _hl = lane & 7
    ZPERM = (((_hl & 1) << 3) | (((_hl >> 1) & 1) << 2)
             | (((_hl >> 2) & 1) << 1))
    SPLATS = [jnp.full((DH,), _inv(h), jnp.int32) for h in range(H)]
    M8 = lane < 8
    M4 = (lane & 4) == 0
    M2 = (lane & 2) == 0
    ZERO16 = jnp.zeros((DH,), jnp.float32)

    _dnums = lax.GatherDimensionNumbers(
        offset_dims=(), collapsed_slice_dims=(0,), start_index_map=(0,))

    def _take(x, idx):
        return lax.gather(x, idx[:, None], _dnums, (1,),
                          mode=lax.GatherScatterMode.PROMISE_IN_BOUNDS)

    def _issue_super(b):
        pltpu.async_copy(idx_hbm.at[pl.ds(ch0 + b * SUP, SUP)],
                         idx2.at[lax.rem(b, 3)], gi)

    def _wait_super():
        pltpu.make_async_copy(idx_hbm.at[pl.ds(0, SUP)], idx2.at[0],
                              gi).wait()

    def _issue_gathers(j):
        b = j // SUP
        bi = j - b * SUP
        sl = lax.rem(b, 3)
        s = j & 1
        pltpu.async_copy(k_hbm.at[idx2.at[sl, bi, 0]], kbuf2.at[s], gkq.at[s])
        pltpu.async_copy(q_hbm.at[idx2.at[sl, bi, 1]], qbuf2.at[s], gkq.at[s])
        pltpu.async_copy(v_hbm.at[idx2.at[sl, bi, 0]], msg4.at[j & 3],
                         gv.at[s])

    _issue_super(0)
    _wait_super()
    _issue_super(1)
    _issue_gathers(0)
    _issue_gathers(1)

    def _chunk(j, carry):
        s = j & 1
        sm = j & 3
        zm = lax.rem(j, 3)
        b = j // SUP
        bi = j - b * SUP
        sl = lax.rem(b, 3)
        pltpu.make_async_copy(k_hbm.at[pl.ds(0, C)], kbuf2.at[s],
                              gkq.at[s]).wait()
        pltpu.make_async_copy(k_hbm.at[pl.ds(0, C)], qbuf2.at[s],
                              gkq.at[s]).wait()
        pltpu.make_async_copy(k_hbm.at[pl.ds(0, C)], msg4.at[sm],
                              gv.at[s]).wait()

        dv16 = idx2[sl, bi, 1, :]

        @plsc.parallel_loop(0, C, 1, unroll=16)
        def _edge(i):
            p = [kbuf2[s, i, pl.ds(h * DH, DH)] * qbuf2[s, i, pl.ds(h * DH, DH)]
                 for h in range(H)]
            a = [ph + _take(ph, X8) for ph in p]
            b_ = [jnp.where(M8, a[2 * k], a[2 * k + 1]) for k in range(4)]
            c = [bk + _take(bk, X4) for bk in b_]
            d = [jnp.where(M4, c[0], c[1]), jnp.where(M4, c[2], c[3])]
            e = [dm + _take(dm, X2) for dm in d]
            f = jnp.where(M2, e[0], e[1])
            g = f + _take(f, X1)
            ev = jnp.exp(jnp.clip(g * SCALE, -5.0, 5.0))
            zvec = _take(ev, ZPERM)
            dvi = _take(dv16, jnp.full((DH,), i, jnp.int32))
            dcol = dvi & 7
            zvi = lax.bitcast_convert_type(zvec, jnp.int32)
            for jj in range(8):
                m = ((dcol ^ jj) - 1) >> 31
                zb3[zm, i, pl.ds(jj * DH, DH)] = lax.bitcast_convert_type(
                    zvi & m, jnp.float32)
            for h in range(H):
                slh = pl.ds(h * DH, DH)
                msg4[sm, i, slh] = msg4[sm, i, slh] * _take(ev, SPLATS[h])

        @pl.when(j >= 2)
        def _wait_scat():
            pltpu.make_async_copy(msg4.at[0], acc_s.at[pl.ds(0, C)],
                                  gs.at[s]).wait()
            pltpu.make_async_copy(zb3.at[0], acc_s.at[pl.ds(0, C)],
                                  gs.at[s]).wait()

        pltpu.async_copy(msg4.at[sm], acc_s.at[idx2.at[sl, bi, 1]],
                         gs.at[s], add=True)
        pltpu.async_copy(zb3.at[zm], acc_s.at[idx2.at[sl, bi, 2]],
                         gs.at[s], add=True)

        @pl.when(j < CH_PER_W - 2)
        def _prefetch():
            jn = j + 2
            bn = jn // SUP

            @pl.when(jn - bn * SUP == 0)
            def _ld():
                _wait_super()

                @pl.when(bn + 1 < NSUP)
                def _nx():
                    _issue_super(bn + 1)

            _issue_gathers(jn)

        return carry

    lax.fori_loop(0, CH_PER_W, _chunk, 0)

    for jj in (CH_PER_W - 2, CH_PER_W - 1):
        pltpu.make_async_copy(msg4.at[0], acc_s.at[pl.ds(0, C)],
                              gs.at[jj & 1]).wait()
        pltpu.make_async_copy(zb3.at[0], acc_s.at[pl.ds(0, C)],
                              gs.at[jj & 1]).wait()

    plsc.subcore_barrier()

    def _bounce(out_ref, acc_r0, out_r0, nrows):
        pltpu.sync_copy(acc_s.at[pl.ds(acc_r0, nrows)],
                        msg4.at[0, pl.ds(0, nrows)])
        pltpu.sync_copy(msg4.at[0, pl.ds(0, nrows)],
                        out_ref.at[cid, pl.ds(out_r0, nrows)])

    row0 = sid * ROWS_PER_TILE

    @pl.when(sid < NS - 1)
    def _copy_full():
        for r in range(ROWS_PER_TILE // C):
            _bounce(wv_out, row0 + r * C, row0 + r * C, C)

    @pl.when(sid == NS - 1)
    def _copy_tail():
        tail0 = (NS - 1) * ROWS_PER_TILE
        for r in range((N - tail0) // C):
            _bounce(wv_out, tail0 + r * C, tail0 + r * C, C)

    zrow0 = sid * ZROWS_PER_TILE
    for r in range(ZROWS_PER_TILE // C):
        _bounce(z_out, ZBASE + zrow0 + r * C, zrow0 + r * C, C)


@functools.partial(jax.jit)
def _edge_call(q, k, v, idx_all):
    mesh = plsc.VectorSubcoreMesh(core_axis_name="c", subcore_axis_name="s")
    fn = pl.kernel(
        _edge_body,
        mesh=mesh,
        out_type=(jax.ShapeDtypeStruct((NC, N, D), jnp.float32),
                  jax.ShapeDtypeStruct((NC, ZROWS, D), jnp.float32)),
        scratch_types=[
            pltpu.VMEM_SHARED((NPAD + ZROWS, D), jnp.float32),
            pltpu.VMEM((3, SUP, 3, C), jnp.int32),
            pltpu.VMEM((2, C, D), jnp.float32),
            pltpu.VMEM((2, C, D), jnp.float32),
            pltpu.VMEM((4, C, D), jnp.float32),
            pltpu.VMEM((3, C, D), jnp.float32),
            pltpu.SemaphoreType.DMA((2,)),
            pltpu.SemaphoreType.DMA((2,)),
            pltpu.SemaphoreType.DMA((2,)),
            pltpu.SemaphoreType.DMA,
        ],
    )
    return fn(q, k, v, idx_all)


_S_np = np.zeros((DH, D), np.float32)
for _h in range(H):
    _S_np[_h, _h * DH:(_h + 1) * DH] = 1.0


def _comb_body(wv_ref, z_ref, s_ref, o_ref):
    wv = wv_ref[0] + wv_ref[1]
    z = z_ref[0] + z_ref[1]
    zb = jnp.dot(z, s_ref[...], preferred_element_type=jnp.float32,
                 precision=lax.Precision.HIGHEST)
    o_ref[...] = wv / (zb + 1e-6)


def _combine(wvp, zp):
    grid = (N // _ROWB,)
    return pl.pallas_call(
        _comb_body,
        grid=grid,
        in_specs=[pl.BlockSpec((NC, _ROWB, D), lambda i: (0, i, 0)),
                  pl.BlockSpec((NC, _ROWB, DH), lambda i: (0, i, 0)),
                  pl.BlockSpec((DH, D), lambda i: (0, 0))],
        out_specs=pl.BlockSpec((_ROWB, D), lambda i: (i, 0)),
        out_shape=jax.ShapeDtypeStruct((N, D), jnp.float32),
    )(wvp, zp, jnp.asarray(_S_np))


def kernel(x, edge_index, WQ, WK, WV):
    q, k, v = _qkv(x, WQ, WK, WV)
    src = edge_index[0]
    dst = edge_index[1]
    zidx = (dst >> 3) + ZBASE
    idx_all = (jnp.stack([src, dst, zidx], axis=0)
               .reshape(3, NCHUNKS, C).transpose(1, 0, 2))
    wvp, zpack = _edge_call(q, k, v, idx_all)
    zp = zpack.reshape(NC, NPAD, DH)[:, :N, :]
    return _combine(wvp, zp)
